# Initial kernel scaffold; baseline (speedup 1.0000x reference)
import jax
import jax.numpy as jnp
from jax.experimental import pallas as pl


def kernel(inputs):
    raise NotImplementedError("write your pallas kernel here")


# speedup vs baseline: 2.9062x; 2.9062x over previous
import jax
import jax.numpy as jnp
from jax import lax
from jax.experimental import pallas as pl

K_OUT = 64


def _topk_body(x_ref, o_ref):
    x = x_ref[...]
    n_rows, n = x.shape
    col = lax.broadcasted_iota(jnp.int32, (n_rows, n), 1)
    kcol = lax.broadcasted_iota(jnp.int32, (n_rows, K_OUT), 1)
    out0 = jnp.zeros((n_rows, K_OUT), jnp.float32)

    def body(k, carry):
        x, out = carry
        m = jnp.max(x, axis=1, keepdims=True)
        out = jnp.where(kcol == k, m, out)
        first = jnp.min(jnp.where(x == m, col, n), axis=1, keepdims=True)
        x = jnp.where(col == first, -jnp.inf, x)
        return x, out

    _, out = lax.fori_loop(0, K_OUT, body, (x, out0))
    o_ref[...] = out


def kernel(inputs):
    return pl.pallas_call(
        _topk_body,
        out_shape=jax.ShapeDtypeStruct((inputs.shape[0], K_OUT), jnp.float32),
    )(inputs)
